# baseline (speedup 1.0000x reference)
import functools

import jax
import jax.numpy as jnp
from jax import lax
from jax.experimental import pallas as pl
from jax.experimental.pallas import tpu as pltpu
from jax.experimental.pallas import tpu_sc as plsc

NC = 2
NS = 16
L = 16

E = 400000
NROWS = 50000
D = 128

EPW = 25024
EPAD = EPW * NS
BLK = 6256
NBLK = EPW // BLK
NITB = BLK // L
CR = 6272
NPASS = 4
ZR = 392
DUMP = 0
NBROWS = 50
B = 128


def _segsum_body(rows_hbm, cols_hbm, vals_hbm, table_hbm, out_hbm,
                 ebr, ebc, ebv, cidx, cval, clid, gbuf, gbuf1,
                 acc, sem, sem1):
  cid = lax.axis_index("c")
  sid = lax.axis_index("s")
  base = sid * EPW

  zero = jnp.zeros((L,), jnp.float32)

  def _zero_row(r, _):
    for k in range(D // L):
      gbuf[r, pl.ds(k * L, L)] = zero
    return 0

  izero = jnp.zeros((L,), jnp.int32)
  idump = jnp.full((L,), DUMP, jnp.int32)
  ones = jnp.ones((L,), jnp.bool_)

  def _pass(p, _):
    chunk = p * NC + cid
    lo = pl.multiple_of(chunk * CR, 8)
    cvr = jnp.minimum(lo + CR, NROWS) - lo

    lax.fori_loop(0, B, _zero_row, 0)
    zbase = pl.multiple_of(jnp.minimum(sid * ZR, CR - ZR), 8)
    for k in range(ZR // B):
      pltpu.sync_copy(gbuf, acc.at[pl.ds(zbase + k * B, B)])
    pltpu.sync_copy(gbuf.at[pl.ds(0, ZR % B)],
                    acc.at[pl.ds(zbase + (ZR // B) * B, ZR % B)])
    plsc.subcore_barrier()

    def _compact(it, cnt):
      sl = pl.ds(it * L, L)
      r = ebr[sl]
      c = ebc[sl]
      v = ebv[sl]
      m = (r >= lo) & (r < lo + CR)
      pos = cnt + plsc.cumsum(m.astype(jnp.int32)) - 1
      pr = pos >> 7
      pc_ = pos & 127
      plsc.store_scatter(cidx, [pr, pc_], c, mask=m)
      plsc.store_scatter(cval, [pr, pc_], v, mask=m)
      plsc.store_scatter(clid, [pr, pc_], r - lo, mask=m)
      pc = plsc.all_reduce_population_count(m)
      return cnt + pc[0]

    def _process(jj, buf):
      def _scale(g, _):
        vv = cval[jj, pl.ds(g * L, L)]
        for r16 in range(L):
          r = g * L + r16
          v = jnp.broadcast_to(vv[r16], (L,))
          for k in range(D // L):
            s = pl.ds(k * L, L)
            buf[r, s] = buf[r, s] * v
        return 0

      lax.fori_loop(0, B // L, _scale, 0)
      pltpu.sync_copy(buf, acc.at[clid.at[jj]], add=True)

    def _block(blk, _):
      off = base + blk * BLK
      pltpu.sync_copy(rows_hbm.at[pl.ds(off, BLK)], ebr)
      pltpu.sync_copy(cols_hbm.at[pl.ds(off, BLK)], ebc)
      pltpu.sync_copy(vals_hbm.at[pl.ds(off, BLK)], ebv)

      cnt = lax.fori_loop(0, NITB, _compact, jnp.int32(0))

      for k in range(B // L):
        pos = cnt + k * L + lax.iota(jnp.int32, L)
        pr = pos >> 7
        pc_ = pos & 127
        plsc.store_scatter(cidx, [pr, pc_], izero, mask=ones)
        plsc.store_scatter(cval, [pr, pc_], zero, mask=ones)
        plsc.store_scatter(clid, [pr, pc_], idump, mask=ones)

      nb = (cnt + (B - 1)) // B

      def _drain(j, _):
        @pl.when(j < nb)
        def _start():
          @pl.when((j & 1) == 0)
          def _():
            pltpu.async_copy(table_hbm.at[cidx.at[j]], gbuf, sem)

          @pl.when((j & 1) == 1)
          def _():
            pltpu.async_copy(table_hbm.at[cidx.at[j]], gbuf1, sem1)

        @pl.when(j > 0)
        def _finish():
          jj = j - 1

          @pl.when((jj & 1) == 0)
          def _():
            pltpu.make_async_copy(table_hbm.at[cidx.at[jj]], gbuf,
                                  sem).wait()
            _process(jj, gbuf)

          @pl.when((jj & 1) == 1)
          def _():
            pltpu.make_async_copy(table_hbm.at[cidx.at[jj]], gbuf1,
                                  sem1).wait()
            _process(jj, gbuf1)

        return 0

      lax.fori_loop(0, nb + 1, _drain, 0)
      return 0

    lax.fori_loop(0, NBLK, _block, 0)
    plsc.subcore_barrier()

    fbase = pl.multiple_of(jnp.minimum(sid * ZR, cvr - ZR), 8)
    pltpu.sync_copy(acc.at[pl.ds(fbase, ZR)],
                    out_hbm.at[pl.ds(lo + fbase, ZR)])
    plsc.subcore_barrier()
    return 0

  lax.fori_loop(0, NPASS, _pass, 0)


def _segment_sum(rows, cols, vals, table):
  pad = EPAD - E
  rows = jnp.concatenate([rows, jnp.zeros((pad,), jnp.int32)])
  cols = jnp.concatenate([cols, jnp.zeros((pad,), jnp.int32)])
  vals = jnp.concatenate([vals, jnp.zeros((pad,), jnp.float32)])
  mesh = plsc.VectorSubcoreMesh(core_axis_name="c", subcore_axis_name="s",
                                num_cores=NC, num_subcores=NS)
  f = pl.kernel(
      _segsum_body,
      out_type=jax.ShapeDtypeStruct((NROWS, D), jnp.float32),
      mesh=mesh,
      scratch_types=[
          pltpu.VMEM((BLK,), jnp.int32),
          pltpu.VMEM((BLK,), jnp.int32),
          pltpu.VMEM((BLK,), jnp.float32),
          pltpu.VMEM((NBROWS, B), jnp.int32),
          pltpu.VMEM((NBROWS, B), jnp.float32),
          pltpu.VMEM((NBROWS, B), jnp.int32),
          pltpu.VMEM((B, D), jnp.float32),
          pltpu.VMEM((B, D), jnp.float32),
          pltpu.VMEM_SHARED((CR, D), jnp.float32),
          pltpu.SemaphoreType.DMA,
          pltpu.SemaphoreType.DMA,
      ],
      compiler_params=pltpu.CompilerParams(needs_layout_passes=False),
      name="coo_segment_sum",
  )
  return f(rows, cols, vals, table)


MM_BLK = 1000


def _linear_body(uu0, uu1, ii0, ii1, w0t, w1t, bb, u_out, i_out):
  bias = bb[0:1, :]
  u = jnp.dot(jnp.maximum(uu0[...], 0.0), w0t[...],
              preferred_element_type=jnp.float32)
  u += jnp.dot(jnp.maximum(uu1[...], 0.0), w1t[...],
               preferred_element_type=jnp.float32)
  u_out[...] = u + bias
  i = jnp.dot(jnp.maximum(ii0[...], 0.0), w0t[...],
              preferred_element_type=jnp.float32)
  i += jnp.dot(jnp.maximum(ii1[...], 0.0), w1t[...],
               preferred_element_type=jnp.float32)
  i_out[...] = i + bias


def _linear(uu0, uu1, ii0, ii1, W, b):
  w0t = jnp.transpose(W[:, :D])
  w1t = jnp.transpose(W[:, D:])
  bb = jnp.broadcast_to(b.reshape(1, D), (8, D))
  blk = pl.BlockSpec((MM_BLK, D), lambda i: (i, 0))
  wblk = pl.BlockSpec((D, D), lambda i: (0, 0))
  bblk = pl.BlockSpec((8, D), lambda i: (0, 0))
  return pl.pallas_call(
      _linear_body,
      grid=(NROWS // MM_BLK,),
      in_specs=[blk, blk, blk, blk, wblk, wblk, bblk],
      out_specs=[blk, blk],
      out_shape=[jax.ShapeDtypeStruct((NROWS, D), jnp.float32),
                 jax.ShapeDtypeStruct((NROWS, D), jnp.float32)],
  )(uu0, uu1, ii0, ii1, w0t, w1t, bb)


def kernel(uis_row_0, uis_col_0, uis_val_0, uis_row_1, uis_col_1, uis_val_1,
           ius_row_0, ius_col_0, ius_val_0, ius_row_1, ius_col_1, ius_val_1,
           u, i, emb_i_0, emb_i_1, emb_u_0, emb_u_1, W, b):
  uu0 = _segment_sum(uis_row_0, uis_col_0, uis_val_0, emb_i_0)
  uu1 = _segment_sum(uis_row_1, uis_col_1, uis_val_1, emb_i_1)
  ii0 = _segment_sum(ius_row_0, ius_col_0, ius_val_0, emb_u_0)
  ii1 = _segment_sum(ius_row_1, ius_col_1, ius_val_1, emb_u_1)
  u_out, i_out = _linear(uu0, uu1, ii0, ii1, W, b)
  return (u_out, i_out)


# speedup vs baseline: 1.1794x; 1.1794x over previous
import functools

import jax
import jax.numpy as jnp
from jax import lax
from jax.experimental import pallas as pl
from jax.experimental.pallas import tpu as pltpu
from jax.experimental.pallas import tpu_sc as plsc

NC = 2
NS = 16
L = 16

E = 400000
NROWS = 50000
D = 128

EPW = 25088
EPAD = EPW * NS
BLK = 3584
NBLK = EPW // BLK
NITB = BLK // L
CR = 8336
NPASS = 3
ZR = 528
DUMP = 0
NBROWS = 29
B = 128


def _segsum_body(rows_hbm, cols_hbm, vals_hbm, table_hbm, out_hbm,
                 ebr, ebc, ebv, cidx, cval, clid, gbuf, gbuf1,
                 acc, sem, sem1):
  cid = lax.axis_index("c")
  sid = lax.axis_index("s")
  base = sid * EPW

  zero = jnp.zeros((L,), jnp.float32)

  def _zero_row(r, _):
    for k in range(D // L):
      gbuf[r, pl.ds(k * L, L)] = zero
    return 0

  izero = jnp.zeros((L,), jnp.int32)
  idump = jnp.full((L,), DUMP, jnp.int32)
  ones = jnp.ones((L,), jnp.bool_)

  def _pass(p, _):
    chunk = p * NC + cid
    lo = pl.multiple_of(chunk * CR, 8)
    cvr = jnp.minimum(lo + CR, NROWS) - lo

    lax.fori_loop(0, B, _zero_row, 0)
    zbase = pl.multiple_of(jnp.minimum(sid * ZR, CR - ZR), 8)
    for k in range(ZR // B):
      pltpu.sync_copy(gbuf, acc.at[pl.ds(zbase + k * B, B)])
    pltpu.sync_copy(gbuf.at[pl.ds(0, ZR % B)],
                    acc.at[pl.ds(zbase + (ZR // B) * B, ZR % B)])
    plsc.subcore_barrier()

    def _compact(it, cnt):
      sl = pl.ds(it * L, L)
      r = ebr[sl]
      c = ebc[sl]
      v = ebv[sl]
      m = (r >= lo) & (r < lo + CR)
      pos = cnt + plsc.cumsum(m.astype(jnp.int32)) - 1
      pr = pos >> 7
      pc_ = pos & 127
      plsc.store_scatter(cidx, [pr, pc_], c, mask=m)
      plsc.store_scatter(cval, [pr, pc_], v, mask=m)
      plsc.store_scatter(clid, [pr, pc_], r - lo, mask=m)
      pc = plsc.all_reduce_population_count(m)
      return cnt + pc[0]

    def _process(jj, buf):
      def _scale(g, _):
        vv = cval[jj, pl.ds(g * L, L)]
        for r16 in range(L):
          r = g * L + r16
          v = jnp.broadcast_to(vv[r16], (L,))
          for k in range(D // L):
            s = pl.ds(k * L, L)
            buf[r, s] = buf[r, s] * v
        return 0

      lax.fori_loop(0, B // L, _scale, 0)
      pltpu.sync_copy(buf, acc.at[clid.at[jj]], add=True)

    def _block(blk, _):
      off = base + blk * BLK
      pltpu.sync_copy(rows_hbm.at[pl.ds(off, BLK)], ebr)
      pltpu.sync_copy(cols_hbm.at[pl.ds(off, BLK)], ebc)
      pltpu.sync_copy(vals_hbm.at[pl.ds(off, BLK)], ebv)

      cnt = lax.fori_loop(0, NITB, _compact, jnp.int32(0))

      for k in range(B // L):
        pos = cnt + k * L + lax.iota(jnp.int32, L)
        pr = pos >> 7
        pc_ = pos & 127
        plsc.store_scatter(cidx, [pr, pc_], izero, mask=ones)
        plsc.store_scatter(cval, [pr, pc_], zero, mask=ones)
        plsc.store_scatter(clid, [pr, pc_], idump, mask=ones)

      nb = (cnt + (B - 1)) // B

      def _drain(j, _):
        @pl.when(j < nb)
        def _start():
          @pl.when((j & 1) == 0)
          def _():
            pltpu.async_copy(table_hbm.at[cidx.at[j]], gbuf, sem)

          @pl.when((j & 1) == 1)
          def _():
            pltpu.async_copy(table_hbm.at[cidx.at[j]], gbuf1, sem1)

        @pl.when(j > 0)
        def _finish():
          jj = j - 1

          @pl.when((jj & 1) == 0)
          def _():
            pltpu.make_async_copy(table_hbm.at[cidx.at[jj]], gbuf,
                                  sem).wait()
            _process(jj, gbuf)

          @pl.when((jj & 1) == 1)
          def _():
            pltpu.make_async_copy(table_hbm.at[cidx.at[jj]], gbuf1,
                                  sem1).wait()
            _process(jj, gbuf1)

        return 0

      lax.fori_loop(0, nb + 1, _drain, 0)
      return 0

    lax.fori_loop(0, NBLK, _block, 0)
    plsc.subcore_barrier()

    fbase = pl.multiple_of(jnp.minimum(sid * ZR, cvr - ZR), 8)
    pltpu.sync_copy(acc.at[pl.ds(fbase, ZR)],
                    out_hbm.at[pl.ds(lo + fbase, ZR)])
    plsc.subcore_barrier()
    return 0

  lax.fori_loop(0, NPASS, _pass, 0)


def _segment_sum(rows, cols, vals, table):
  pad = EPAD - E
  rows = jnp.concatenate([rows, jnp.zeros((pad,), jnp.int32)])
  cols = jnp.concatenate([cols, jnp.zeros((pad,), jnp.int32)])
  vals = jnp.concatenate([vals, jnp.zeros((pad,), jnp.float32)])
  mesh = plsc.VectorSubcoreMesh(core_axis_name="c", subcore_axis_name="s",
                                num_cores=NC, num_subcores=NS)
  f = pl.kernel(
      _segsum_body,
      out_type=jax.ShapeDtypeStruct((NROWS, D), jnp.float32),
      mesh=mesh,
      scratch_types=[
          pltpu.VMEM((BLK,), jnp.int32),
          pltpu.VMEM((BLK,), jnp.int32),
          pltpu.VMEM((BLK,), jnp.float32),
          pltpu.VMEM((NBROWS, B), jnp.int32),
          pltpu.VMEM((NBROWS, B), jnp.float32),
          pltpu.VMEM((NBROWS, B), jnp.int32),
          pltpu.VMEM((B, D), jnp.float32),
          pltpu.VMEM((B, D), jnp.float32),
          pltpu.VMEM_SHARED((CR, D), jnp.float32),
          pltpu.SemaphoreType.DMA,
          pltpu.SemaphoreType.DMA,
      ],
      compiler_params=pltpu.CompilerParams(needs_layout_passes=False),
      name="coo_segment_sum",
  )
  return f(rows, cols, vals, table)


MM_BLK = 1000


def _linear_body(uu0, uu1, ii0, ii1, w0t, w1t, bb, u_out, i_out):
  bias = bb[0:1, :]
  u = jnp.dot(jnp.maximum(uu0[...], 0.0), w0t[...],
              preferred_element_type=jnp.float32)
  u += jnp.dot(jnp.maximum(uu1[...], 0.0), w1t[...],
               preferred_element_type=jnp.float32)
  u_out[...] = u + bias
  i = jnp.dot(jnp.maximum(ii0[...], 0.0), w0t[...],
              preferred_element_type=jnp.float32)
  i += jnp.dot(jnp.maximum(ii1[...], 0.0), w1t[...],
               preferred_element_type=jnp.float32)
  i_out[...] = i + bias


def _linear(uu0, uu1, ii0, ii1, W, b):
  w0t = jnp.transpose(W[:, :D])
  w1t = jnp.transpose(W[:, D:])
  bb = jnp.broadcast_to(b.reshape(1, D), (8, D))
  blk = pl.BlockSpec((MM_BLK, D), lambda i: (i, 0))
  wblk = pl.BlockSpec((D, D), lambda i: (0, 0))
  bblk = pl.BlockSpec((8, D), lambda i: (0, 0))
  return pl.pallas_call(
      _linear_body,
      grid=(NROWS // MM_BLK,),
      in_specs=[blk, blk, blk, blk, wblk, wblk, bblk],
      out_specs=[blk, blk],
      out_shape=[jax.ShapeDtypeStruct((NROWS, D), jnp.float32),
                 jax.ShapeDtypeStruct((NROWS, D), jnp.float32)],
  )(uu0, uu1, ii0, ii1, w0t, w1t, bb)


def kernel(uis_row_0, uis_col_0, uis_val_0, uis_row_1, uis_col_1, uis_val_1,
           ius_row_0, ius_col_0, ius_val_0, ius_row_1, ius_col_1, ius_val_1,
           u, i, emb_i_0, emb_i_1, emb_u_0, emb_u_1, W, b):
  uu0 = _segment_sum(uis_row_0, uis_col_0, uis_val_0, emb_i_0)
  uu1 = _segment_sum(uis_row_1, uis_col_1, uis_val_1, emb_i_1)
  ii0 = _segment_sum(ius_row_0, ius_col_0, ius_val_0, emb_u_0)
  ii1 = _segment_sum(ius_row_1, ius_col_1, ius_val_1, emb_u_1)
  u_out, i_out = _linear(uu0, uu1, ii0, ii1, W, b)
  return (u_out, i_out)


# speedup vs baseline: 1.2636x; 1.0714x over previous
import functools

import jax
import jax.numpy as jnp
from jax import lax
from jax.experimental import pallas as pl
from jax.experimental.pallas import tpu as pltpu
from jax.experimental.pallas import tpu_sc as plsc

NC = 2
NS = 16
L = 16

E = 400000
NROWS = 50000
D = 128

EPW = 25024
EPAD = EPW * NS
BLK = 6256
NBLK = EPW // BLK
NITB = BLK // L
CR = 8336
NPASS = 3
ZR = 528
DUMP = 0
NBROWS = 50
B = 128


def _segsum_body(rows_hbm, cols_hbm, vals_hbm, table_hbm, out_hbm,
                 ebr, ebc, ebv, cidx, cval, clid, gbuf,
                 acc, sem):
  cid = lax.axis_index("c")
  sid = lax.axis_index("s")
  base = sid * EPW

  zero = jnp.zeros((L,), jnp.float32)

  def _zero_row(r, _):
    for k in range(D // L):
      gbuf[r, pl.ds(k * L, L)] = zero
    return 0

  izero = jnp.zeros((L,), jnp.int32)
  idump = jnp.full((L,), DUMP, jnp.int32)
  ones = jnp.ones((L,), jnp.bool_)

  def _pass(p, _):
    chunk = p * NC + cid
    lo = pl.multiple_of(chunk * CR, 8)
    cvr = jnp.minimum(lo + CR, NROWS) - lo

    lax.fori_loop(0, B, _zero_row, 0)
    zbase = pl.multiple_of(jnp.minimum(sid * ZR, CR - ZR), 8)
    for k in range(ZR // B):
      pltpu.sync_copy(gbuf, acc.at[pl.ds(zbase + k * B, B)])
    pltpu.sync_copy(gbuf.at[pl.ds(0, ZR % B)],
                    acc.at[pl.ds(zbase + (ZR // B) * B, ZR % B)])
    plsc.subcore_barrier()

    def _compact(it, cnt):
      sl = pl.ds(it * L, L)
      r = ebr[sl]
      c = ebc[sl]
      v = ebv[sl]
      m = (r >= lo) & (r < lo + CR)
      pos = cnt + plsc.cumsum(m.astype(jnp.int32)) - 1
      pr = pos >> 7
      pc_ = pos & 127
      plsc.store_scatter(cidx, [pr, pc_], c, mask=m)
      plsc.store_scatter(cval, [pr, pc_], v, mask=m)
      plsc.store_scatter(clid, [pr, pc_], r - lo, mask=m)
      pc = plsc.all_reduce_population_count(m)
      return cnt + pc[0]

    def _batch(j, _):
      pltpu.async_copy(table_hbm.at[cidx.at[j]], gbuf, sem).wait()

      def _scale(g, _):
        vv = cval[j, pl.ds(g * L, L)]
        for r16 in range(L):
          r = g * L + r16
          v = jnp.broadcast_to(vv[r16], (L,))
          for k in range(D // L):
            s = pl.ds(k * L, L)
            gbuf[r, s] = gbuf[r, s] * v
        return 0

      lax.fori_loop(0, B // L, _scale, 0)
      pltpu.sync_copy(gbuf, acc.at[clid.at[j]], add=True)
      return 0

    def _block(blk, _):
      off = base + blk * BLK
      pltpu.sync_copy(rows_hbm.at[pl.ds(off, BLK)], ebr)
      pltpu.sync_copy(cols_hbm.at[pl.ds(off, BLK)], ebc)
      pltpu.sync_copy(vals_hbm.at[pl.ds(off, BLK)], ebv)

      cnt = lax.fori_loop(0, NITB, _compact, jnp.int32(0))

      for k in range(B // L):
        pos = cnt + k * L + lax.iota(jnp.int32, L)
        pr = pos >> 7
        pc_ = pos & 127
        plsc.store_scatter(cidx, [pr, pc_], izero, mask=ones)
        plsc.store_scatter(cval, [pr, pc_], zero, mask=ones)
        plsc.store_scatter(clid, [pr, pc_], idump, mask=ones)

      nb = (cnt + (B - 1)) // B
      lax.fori_loop(0, nb, _batch, 0)
      return 0

    lax.fori_loop(0, NBLK, _block, 0)
    plsc.subcore_barrier()

    fbase = pl.multiple_of(jnp.minimum(sid * ZR, cvr - ZR), 8)
    pltpu.sync_copy(acc.at[pl.ds(fbase, ZR)],
                    out_hbm.at[pl.ds(lo + fbase, ZR)])
    plsc.subcore_barrier()
    return 0

  lax.fori_loop(0, NPASS, _pass, 0)


def _segment_sum(rows, cols, vals, table):
  pad = EPAD - E
  rows = jnp.concatenate([rows, jnp.zeros((pad,), jnp.int32)])
  cols = jnp.concatenate([cols, jnp.zeros((pad,), jnp.int32)])
  vals = jnp.concatenate([vals, jnp.zeros((pad,), jnp.float32)])
  mesh = plsc.VectorSubcoreMesh(core_axis_name="c", subcore_axis_name="s",
                                num_cores=NC, num_subcores=NS)
  f = pl.kernel(
      _segsum_body,
      out_type=jax.ShapeDtypeStruct((NROWS, D), jnp.float32),
      mesh=mesh,
      scratch_types=[
          pltpu.VMEM((BLK,), jnp.int32),
          pltpu.VMEM((BLK,), jnp.int32),
          pltpu.VMEM((BLK,), jnp.float32),
          pltpu.VMEM((NBROWS, B), jnp.int32),
          pltpu.VMEM((NBROWS, B), jnp.float32),
          pltpu.VMEM((NBROWS, B), jnp.int32),
          pltpu.VMEM((B, D), jnp.float32),
          pltpu.VMEM_SHARED((CR, D), jnp.float32),
          pltpu.SemaphoreType.DMA,
      ],
      compiler_params=pltpu.CompilerParams(needs_layout_passes=False),
      name="coo_segment_sum",
  )
  return f(rows, cols, vals, table)


MM_BLK = 1000


def _linear_body(uu0, uu1, ii0, ii1, w0t, w1t, bb, u_out, i_out):
  bias = bb[0:1, :]
  u = jnp.dot(jnp.maximum(uu0[...], 0.0), w0t[...],
              preferred_element_type=jnp.float32)
  u += jnp.dot(jnp.maximum(uu1[...], 0.0), w1t[...],
               preferred_element_type=jnp.float32)
  u_out[...] = u + bias
  i = jnp.dot(jnp.maximum(ii0[...], 0.0), w0t[...],
              preferred_element_type=jnp.float32)
  i += jnp.dot(jnp.maximum(ii1[...], 0.0), w1t[...],
               preferred_element_type=jnp.float32)
  i_out[...] = i + bias


def _linear(uu0, uu1, ii0, ii1, W, b):
  w0t = jnp.transpose(W[:, :D])
  w1t = jnp.transpose(W[:, D:])
  bb = jnp.broadcast_to(b.reshape(1, D), (8, D))
  blk = pl.BlockSpec((MM_BLK, D), lambda i: (i, 0))
  wblk = pl.BlockSpec((D, D), lambda i: (0, 0))
  bblk = pl.BlockSpec((8, D), lambda i: (0, 0))
  return pl.pallas_call(
      _linear_body,
      grid=(NROWS // MM_BLK,),
      in_specs=[blk, blk, blk, blk, wblk, wblk, bblk],
      out_specs=[blk, blk],
      out_shape=[jax.ShapeDtypeStruct((NROWS, D), jnp.float32),
                 jax.ShapeDtypeStruct((NROWS, D), jnp.float32)],
  )(uu0, uu1, ii0, ii1, w0t, w1t, bb)


def kernel(uis_row_0, uis_col_0, uis_val_0, uis_row_1, uis_col_1, uis_val_1,
           ius_row_0, ius_col_0, ius_val_0, ius_row_1, ius_col_1, ius_val_1,
           u, i, emb_i_0, emb_i_1, emb_u_0, emb_u_1, W, b):
  uu0 = _segment_sum(uis_row_0, uis_col_0, uis_val_0, emb_i_0)
  uu1 = _segment_sum(uis_row_1, uis_col_1, uis_val_1, emb_i_1)
  ii0 = _segment_sum(ius_row_0, ius_col_0, ius_val_0, emb_u_0)
  ii1 = _segment_sum(ius_row_1, ius_col_1, ius_val_1, emb_u_1)
  u_out, i_out = _linear(uu0, uu1, ii0, ii1, W, b)
  return (u_out, i_out)
